# baseline (speedup 1.0000x reference)
import functools

import numpy as np
import jax
import jax.numpy as jnp
from jax.experimental import pallas as pl
from jax.experimental.pallas import tpu as pltpu

D_MODEL = 512
D_K = 64
D_V = 64
H = 8
_C = 5


def _threefry2x32_np(k1, k2, x1, x2):
    u32 = np.uint32
    def rotl(v, d):
        return ((v << u32(d)) | (v >> u32(32 - d))).astype(u32)
    ks = [u32(k1), u32(k2), u32(k1) ^ u32(k2) ^ u32(0x1BD11BDA)]
    x = [(x1 + ks[0]).astype(u32), (x2 + ks[1]).astype(u32)]
    rotations = ((13, 15, 26, 6), (17, 29, 16, 24))
    for i in range(5):
        for r in rotations[i % 2]:
            x[0] = (x[0] + x[1]).astype(u32)
            x[1] = x[0] ^ rotl(x[1], r)
        x[0] = (x[0] + ks[(i + 1) % 3]).astype(u32)
        x[1] = (x[1] + ks[(i + 2) % 3] + u32(i + 1)).astype(u32)
    return x[0], x[1]


def _randint_np(seed, shape, minval, maxval):
    u32 = np.uint32
    n = int(np.prod(shape))
    b1, b2 = _threefry2x32_np(u32(0), u32(seed),
                              np.zeros(2, u32), np.arange(2, dtype=u32))
    idx = np.arange(n, dtype=np.uint64)
    hi = (idx >> np.uint64(32)).astype(u32)
    lo = idx.astype(u32)

    def rbits(ka, kb):
        a, b = _threefry2x32_np(ka, kb, hi, lo)
        return a ^ b

    higher_bits = rbits(b1[0], b2[0])
    lower_bits = rbits(b1[1], b2[1])
    span = u32(maxval - minval)
    multiplier = u32((2 ** 16) % int(span))
    multiplier = u32((int(multiplier) * int(multiplier)) % int(span))
    with np.errstate(over='ignore'):
        offset = ((higher_bits % span) * multiplier + (lower_bits % span)) % span
    return (np.int32(minval) + offset.astype(np.int32)).reshape(shape)


@functools.lru_cache(maxsize=None)
def _constants(L_Q, L_K):
    u_k = min(int(_C * np.log(L_K)), L_Q)
    u_q = min(int(_C * np.log(L_Q)), L_Q)
    idx = _randint_np(42, (L_Q, u_k), 0, L_K)
    cnt = np.zeros((L_Q, L_K), np.float32)
    np.add.at(cnt, (np.arange(L_Q)[:, None], idx), 1.0)
    tri = np.tril(np.ones((L_K, L_K), np.float32))
    return u_k, u_q, cnt, tri


def _fused_kernel(u_k, u_q, L_Q, L_K,
                  xq_ref, xk_ref, xv_ref, wq_ref, wk_ref, wv_ref,
                  wfc_ref, g_ref, b_ref, cnt_ref, tri_ref, o_ref):
    f32 = jnp.float32
    neg = f32(-jnp.inf)
    xq = xq_ref[0]
    xk = xk_ref[0]
    xv = xv_ref[0]
    cnt = cnt_ref[...]
    supported = cnt > 0
    tri = tri_ref[...]

    m_rows = []
    vals_list = []
    csum_list = []
    for h in range(H):
        Qh = jnp.dot(xq, wq_ref[h], preferred_element_type=f32)
        Kh = jnp.dot(xk, wk_ref[h], preferred_element_type=f32)
        Vh = jnp.dot(xv, wv_ref[h], preferred_element_type=f32)
        S = jnp.dot(Qh, Kh.T, preferred_element_type=f32)
        m_max = jnp.max(jnp.where(supported, S, neg), axis=1, keepdims=True)
        Kbar = jnp.dot(cnt, Kh, preferred_element_type=f32)
        m_mean = jnp.sum(Qh * Kbar, axis=1, keepdims=True) * f32(1.0 / u_k)
        m_rows.append((m_max - m_mean).T)
        Ss = S * f32(1.0 / np.sqrt(D_K))
        e = jnp.exp(Ss - jnp.max(Ss, axis=1, keepdims=True))
        r = f32(1.0) / jnp.sum(e, axis=1, keepdims=True)
        vals_list.append(jnp.dot(e, Vh, preferred_element_type=f32) * r)
        csum_list.append(jnp.dot(tri, Vh, preferred_element_type=f32))

    Mv = jnp.concatenate(m_rows, axis=0)
    iota = jax.lax.broadcasted_iota(jnp.int32, (H, L_Q), 1)
    sel = jnp.zeros((H, L_Q), f32)
    for _ in range(u_q):
        mx = jnp.max(Mv, axis=1, keepdims=True)
        first = jnp.min(jnp.where(Mv == mx, iota, jnp.int32(L_Q)),
                        axis=1, keepdims=True)
        onehot = iota == first
        sel = jnp.maximum(sel, onehot.astype(f32))
        Mv = jnp.where(onehot, neg, Mv)

    acc = None
    for h in range(H):
        sel_h = sel[h:h + 1].T > f32(0.5)
        ctx = jnp.where(sel_h, vals_list[h], csum_list[h])
        part = jnp.dot(ctx, wfc_ref[h], preferred_element_type=f32)
        acc = part if acc is None else acc + part

    x = acc + xq
    mu = jnp.mean(x, axis=1, keepdims=True)
    xc = x - mu
    var = jnp.mean(xc * xc, axis=1, keepdims=True)
    o_ref[0] = xc * jax.lax.rsqrt(var + f32(1e-5)) * g_ref[...] + b_ref[...]


def kernel(input_Q, input_K, input_V, attn_mask, W_Q, W_K, W_V, W_fc,
           ln_gamma, ln_beta):
    B, L_Q, _ = input_Q.shape
    L_K = input_K.shape[1]
    u_k, u_q, cnt_np, tri_np = _constants(L_Q, L_K)
    cnt = jnp.asarray(cnt_np)
    tri = jnp.asarray(tri_np)

    out = pl.pallas_call(
        functools.partial(_fused_kernel, u_k, u_q, L_Q, L_K),
        grid=(B,),
        in_specs=[
            pl.BlockSpec((1, L_Q, D_MODEL), lambda b: (b, 0, 0)),
            pl.BlockSpec((1, L_K, D_MODEL), lambda b: (b, 0, 0)),
            pl.BlockSpec((1, L_K, D_MODEL), lambda b: (b, 0, 0)),
            pl.BlockSpec((H, D_MODEL, D_K), lambda b: (0, 0, 0)),
            pl.BlockSpec((H, D_MODEL, D_K), lambda b: (0, 0, 0)),
            pl.BlockSpec((H, D_MODEL, D_V), lambda b: (0, 0, 0)),
            pl.BlockSpec((H, D_V, D_MODEL), lambda b: (0, 0, 0)),
            pl.BlockSpec((1, D_MODEL), lambda b: (0, 0)),
            pl.BlockSpec((1, D_MODEL), lambda b: (0, 0)),
            pl.BlockSpec((L_Q, L_K), lambda b: (0, 0)),
            pl.BlockSpec((L_K, L_K), lambda b: (0, 0)),
        ],
        out_specs=pl.BlockSpec((1, L_Q, D_MODEL), lambda b: (b, 0, 0)),
        out_shape=jax.ShapeDtypeStruct((B, L_Q, D_MODEL), jnp.float32),
        compiler_params=pltpu.CompilerParams(
            dimension_semantics=("parallel",)),
    )(input_Q, input_K, input_V,
      W_Q.reshape(D_MODEL, H, D_K).transpose(1, 0, 2),
      W_K.reshape(D_MODEL, H, D_K).transpose(1, 0, 2),
      W_V.reshape(D_MODEL, H, D_V).transpose(1, 0, 2),
      W_fc.reshape(H, D_V, D_MODEL),
      ln_gamma.reshape(1, D_MODEL), ln_beta.reshape(1, D_MODEL),
      cnt, tri)
    return out


# speedup vs baseline: 1.0963x; 1.0963x over previous
import functools

import numpy as np
import jax
import jax.numpy as jnp
from jax.experimental import pallas as pl
from jax.experimental.pallas import tpu as pltpu

D_MODEL = 512
D_K = 64
D_V = 64
H = 8
_C = 5


def _threefry2x32_np(k1, k2, x1, x2):
    u32 = np.uint32
    def rotl(v, d):
        return ((v << u32(d)) | (v >> u32(32 - d))).astype(u32)
    ks = [u32(k1), u32(k2), u32(k1) ^ u32(k2) ^ u32(0x1BD11BDA)]
    x = [(x1 + ks[0]).astype(u32), (x2 + ks[1]).astype(u32)]
    rotations = ((13, 15, 26, 6), (17, 29, 16, 24))
    for i in range(5):
        for r in rotations[i % 2]:
            x[0] = (x[0] + x[1]).astype(u32)
            x[1] = x[0] ^ rotl(x[1], r)
        x[0] = (x[0] + ks[(i + 1) % 3]).astype(u32)
        x[1] = (x[1] + ks[(i + 2) % 3] + u32(i + 1)).astype(u32)
    return x[0], x[1]


def _randint_np(seed, shape, minval, maxval):
    u32 = np.uint32
    n = int(np.prod(shape))
    b1, b2 = _threefry2x32_np(u32(0), u32(seed),
                              np.zeros(2, u32), np.arange(2, dtype=u32))
    idx = np.arange(n, dtype=np.uint64)
    hi = (idx >> np.uint64(32)).astype(u32)
    lo = idx.astype(u32)

    def rbits(ka, kb):
        a, b = _threefry2x32_np(ka, kb, hi, lo)
        return a ^ b

    higher_bits = rbits(b1[0], b2[0])
    lower_bits = rbits(b1[1], b2[1])
    span = u32(maxval - minval)
    multiplier = u32((2 ** 16) % int(span))
    multiplier = u32((int(multiplier) * int(multiplier)) % int(span))
    with np.errstate(over='ignore'):
        offset = ((higher_bits % span) * multiplier + (lower_bits % span)) % span
    return (np.int32(minval) + offset.astype(np.int32)).reshape(shape)


@functools.lru_cache(maxsize=None)
def _constants(L_Q, L_K):
    u_k = min(int(_C * np.log(L_K)), L_Q)
    u_q = min(int(_C * np.log(L_Q)), L_Q)
    idx = _randint_np(42, (L_Q, u_k), 0, L_K)
    cnt = np.zeros((L_Q, L_K), np.float32)
    np.add.at(cnt, (np.arange(L_Q)[:, None], idx), 1.0)
    tri = np.tril(np.ones((L_K, L_K), np.float32))
    return u_k, u_q, cnt, tri


def _fused_kernel(u_k, u_q, L_Q, L_K,
                  xq_ref, xk_ref, xv_ref, wq_ref, wk_ref, wv_ref,
                  wfc_ref, g_ref, b_ref, cnt_ref, tri_ref, o_ref):
    f32 = jnp.float32
    neg = f32(-jnp.inf)
    xq = xq_ref[0]
    xk = xk_ref[0]
    xv = xv_ref[0]
    cnt = cnt_ref[...]
    supported = cnt > 0
    tri = tri_ref[...]

    m_rows = []
    vals_list = []
    csum_list = []
    for h in range(H):
        Qh = jnp.dot(xq, wq_ref[h], preferred_element_type=f32)
        Kh = jnp.dot(xk, wk_ref[h], preferred_element_type=f32)
        Vh = jnp.dot(xv, wv_ref[h], preferred_element_type=f32)
        S = jnp.dot(Qh, Kh.T, preferred_element_type=f32)
        m_max = jnp.max(jnp.where(supported, S, neg), axis=1, keepdims=True)
        m_mean = jnp.sum(S * cnt, axis=1, keepdims=True) * f32(1.0 / u_k)
        m_rows.append((m_max - m_mean).T)
        Ss = S * f32(1.0 / np.sqrt(D_K))
        e = jnp.exp(Ss - jnp.max(Ss, axis=1, keepdims=True))
        r = f32(1.0) / jnp.sum(e, axis=1, keepdims=True)
        vals_list.append(jnp.dot(e, Vh, preferred_element_type=f32) * r)
        csum_list.append(jnp.dot(tri, Vh, preferred_element_type=f32))

    Mv = jnp.concatenate(m_rows, axis=0)
    iota = jax.lax.broadcasted_iota(jnp.int32, (H, L_Q), 1)
    sel = jnp.zeros((H, L_Q), f32)
    for _ in range(u_q):
        mx = jnp.max(Mv, axis=1, keepdims=True)
        first = jnp.min(jnp.where(Mv == mx, iota, jnp.int32(L_Q)),
                        axis=1, keepdims=True)
        onehot = iota == first
        sel = jnp.maximum(sel, onehot.astype(f32))
        Mv = jnp.where(onehot, neg, Mv)

    acc = None
    for h in range(H):
        sel_h = sel[h:h + 1].T > f32(0.5)
        ctx = jnp.where(sel_h, vals_list[h], csum_list[h])
        part = jnp.dot(ctx, wfc_ref[h], preferred_element_type=f32)
        acc = part if acc is None else acc + part

    x = acc + xq
    mu = jnp.mean(x, axis=1, keepdims=True)
    xc = x - mu
    var = jnp.mean(xc * xc, axis=1, keepdims=True)
    o_ref[0] = xc * jax.lax.rsqrt(var + f32(1e-5)) * g_ref[...] + b_ref[...]


def kernel(input_Q, input_K, input_V, attn_mask, W_Q, W_K, W_V, W_fc,
           ln_gamma, ln_beta):
    B, L_Q, _ = input_Q.shape
    L_K = input_K.shape[1]
    u_k, u_q, cnt_np, tri_np = _constants(L_Q, L_K)
    cnt = jnp.asarray(cnt_np)
    tri = jnp.asarray(tri_np)

    out = pl.pallas_call(
        functools.partial(_fused_kernel, u_k, u_q, L_Q, L_K),
        grid=(B,),
        in_specs=[
            pl.BlockSpec((1, L_Q, D_MODEL), lambda b: (b, 0, 0)),
            pl.BlockSpec((1, L_K, D_MODEL), lambda b: (b, 0, 0)),
            pl.BlockSpec((1, L_K, D_MODEL), lambda b: (b, 0, 0)),
            pl.BlockSpec((H, D_MODEL, D_K), lambda b: (0, 0, 0)),
            pl.BlockSpec((H, D_MODEL, D_K), lambda b: (0, 0, 0)),
            pl.BlockSpec((H, D_MODEL, D_V), lambda b: (0, 0, 0)),
            pl.BlockSpec((H, D_V, D_MODEL), lambda b: (0, 0, 0)),
            pl.BlockSpec((1, D_MODEL), lambda b: (0, 0)),
            pl.BlockSpec((1, D_MODEL), lambda b: (0, 0)),
            pl.BlockSpec((L_Q, L_K), lambda b: (0, 0)),
            pl.BlockSpec((L_K, L_K), lambda b: (0, 0)),
        ],
        out_specs=pl.BlockSpec((1, L_Q, D_MODEL), lambda b: (b, 0, 0)),
        out_shape=jax.ShapeDtypeStruct((B, L_Q, D_MODEL), jnp.float32),
        compiler_params=pltpu.CompilerParams(
            dimension_semantics=("parallel",)),
    )(input_Q, input_K, input_V,
      W_Q.reshape(D_MODEL, H, D_K).transpose(1, 0, 2),
      W_K.reshape(D_MODEL, H, D_K).transpose(1, 0, 2),
      W_V.reshape(D_MODEL, H, D_V).transpose(1, 0, 2),
      W_fc.reshape(H, D_V, D_MODEL),
      ln_gamma.reshape(1, D_MODEL), ln_beta.reshape(1, D_MODEL),
      cnt, tri)
    return out
